# Initial kernel scaffold; baseline (speedup 1.0000x reference)
import jax
import jax.numpy as jnp
from jax.experimental import pallas as pl


def kernel(inputs, token_table, pos_table):
    raise NotImplementedError("write your pallas kernel here")


# speedup vs baseline: 3.0636x; 3.0636x over previous
import functools

import jax
import jax.numpy as jnp
from jax import lax
from jax.experimental import pallas as pl
from jax.experimental.pallas import tpu as pltpu
from jax.experimental.pallas import tpu_sc as plsc

_B, _L, _D = 4096, 200, 64
_HALF = _L // 2


def _build():
    info = plsc.get_sparse_core_info()
    nc, ns = info.num_cores, info.num_subcores
    nw = nc * ns
    seq_per_w = _B // nw
    mesh = plsc.VectorSubcoreMesh(core_axis_name="c", subcore_axis_name="s")

    @functools.partial(
        pl.kernel,
        mesh=mesh,
        compiler_params=pltpu.CompilerParams(use_tc_tiling_on_sc=False),
        out_type=jax.ShapeDtypeStruct((_B * _L, _D), jnp.float32),
        scratch_types=[
            pltpu.VMEM((2, _HALF), jnp.int32),
            pltpu.VMEM((_L, _D), jnp.float32),
            pltpu.VMEM((_L, _D), jnp.float32),
            pltpu.SemaphoreType.DMA,
        ],
    )
    def k(idx_hbm, tok_hbm, pos_hbm, out_hbm, idx_v, rows_v, pos_v, sem):
        wid = lax.axis_index("s") * nc + lax.axis_index("c")
        pltpu.sync_copy(pos_hbm, pos_v)

        def body(i, carry):
            b = wid * seq_per_w + i
            pltpu.sync_copy(idx_hbm.at[b], idx_v)
            cp0 = pltpu.async_copy(
                tok_hbm.at[idx_v.at[0]], rows_v.at[pl.ds(0, _HALF)], sem)
            cp1 = pltpu.async_copy(
                tok_hbm.at[idx_v.at[1]], rows_v.at[pl.ds(_HALF, _HALF)], sem)
            cp0.wait()
            cp1.wait()

            def add_row(r, c2):
                for j in range(_D // 16):
                    sl = pl.ds(j * 16, 16)
                    rows_v[r, sl] = rows_v[r, sl] + pos_v[r, sl]
                return c2

            lax.fori_loop(0, _L, add_row, 0)
            pltpu.sync_copy(rows_v, out_hbm.at[pl.ds(b * _L, _L)])
            return carry

        lax.fori_loop(0, seq_per_w, body, 0)

    return k


def kernel(inputs, token_table, pos_table):
    idx3 = inputs.reshape(_B, 2, _HALF).astype(jnp.int32)
    out = _build()(idx3, token_table, pos_table)
    return out.reshape(_B, _L, _D)


# speedup vs baseline: 3.9270x; 1.2818x over previous
import functools

import jax
import jax.numpy as jnp
from jax import lax
from jax.experimental import pallas as pl
from jax.experimental.pallas import tpu as pltpu
from jax.experimental.pallas import tpu_sc as plsc

_B, _L, _D = 4096, 200, 64
_HALF = _L // 2
_NB = 4


def _build():
    info = plsc.get_sparse_core_info()
    nc, ns = info.num_cores, info.num_subcores
    nw = nc * ns
    seq_per_w = _B // nw
    mesh = plsc.VectorSubcoreMesh(core_axis_name="c", subcore_axis_name="s")

    @functools.partial(
        pl.kernel,
        mesh=mesh,
        compiler_params=pltpu.CompilerParams(use_tc_tiling_on_sc=False),
        out_type=jax.ShapeDtypeStruct((_B * _L, _D), jnp.float32),
        scratch_types=[
            pltpu.VMEM((seq_per_w, 2, _HALF), jnp.int32),
            pltpu.VMEM((_NB, _L, _D), jnp.float32),
            pltpu.VMEM((_L, _D), jnp.float32),
            pltpu.SemaphoreType.DMA,
            pltpu.SemaphoreType.DMA,
        ],
    )
    def k(idx_hbm, tok_hbm, pos_hbm, out_hbm, idx_v, rows_v, pos_v, sem_g, sem_o):
        wid = lax.axis_index("s") * nc + lax.axis_index("c")
        base_seq = wid * seq_per_w
        pltpu.sync_copy(pos_hbm, pos_v)
        pltpu.sync_copy(idx_hbm.at[pl.ds(base_seq, seq_per_w)], idx_v)

        def issue_gather(i, b):
            for h in range(2):
                pltpu.async_copy(
                    tok_hbm.at[idx_v.at[i, h]],
                    rows_v.at[b, pl.ds(h * _HALF, _HALF)], sem_g)

        def wait_gather(b):
            for h in range(2):
                pltpu.make_async_copy(
                    tok_hbm.at[pl.ds(0, _HALF)],
                    rows_v.at[b, pl.ds(h * _HALF, _HALF)], sem_g).wait()

        def wait_out(b):
            pltpu.make_async_copy(
                rows_v.at[b], out_hbm.at[pl.ds(0, _L)], sem_o).wait()

        for b in range(_NB - 1):
            issue_gather(b, b)

        def outer(t, carry):
            for b in range(_NB):
                i = t * _NB + b
                wait_gather(b)

                @pl.when(i >= 1)
                def _():
                    wait_out((b + _NB - 1) % _NB)

                @pl.when(i + (_NB - 1) < seq_per_w)
                def _():
                    issue_gather(i + (_NB - 1), (b + _NB - 1) % _NB)

                def add_rows(r, c2):
                    for rr in range(2):
                        for j in range(_D // 16):
                            sl = pl.ds(j * 16, 16)
                            rows_v[b, 2 * r + rr, sl] = (
                                rows_v[b, 2 * r + rr, sl] + pos_v[2 * r + rr, sl])
                    return c2

                lax.fori_loop(0, _L // 2, add_rows, 0)
                pltpu.async_copy(
                    rows_v.at[b],
                    out_hbm.at[pl.ds((base_seq + i) * _L, _L)], sem_o)
            return carry

        lax.fori_loop(0, seq_per_w // _NB, outer, 0)
        wait_out(0)

    return k


def kernel(inputs, token_table, pos_table):
    idx3 = inputs.reshape(_B, 2, _HALF).astype(jnp.int32)
    out = _build()(idx3, token_table, pos_table)
    return out.reshape(_B, _L, _D)
